# baseline (speedup 1.0000x reference)
import jax
import jax.numpy as jnp
from jax import lax
from jax.experimental import pallas as pl
from jax.experimental.pallas import tpu as pltpu
from jax.experimental.pallas import tpu_sc as plsc

N = 10000
E = 320000
D = 128

NC = 2
NS = 16
NW = NC * NS
EPW = E // NW
CH = 128
NCHF = E // (CH * NW)
EXTRA = (E // CH) - NCHF * NW
RPT = 624
REM = N - NS * RPT
ZR = 48


def _sc_body(x_hbm, src_hbm, dst_hbm, sum_out, cnt_out,
             acc, is0, is1, is2, is3, id0, id1, id2, id3, rows0, rows1,
             sem_g0, sem_g1, sem_s0, sem_s1, sem_i0, sem_i1, sem_i2, sem_i3,
             sem_w):
    c = lax.axis_index("c")
    s = lax.axis_index("s")
    wid = s * NC + c
    rbase = s * RPT

    idx_s = (is0, is1, is2, is3)
    idx_d = (id0, id1, id2, id3)
    rows = (rows0, rows1)
    sem_g = (sem_g0, sem_g1)
    sem_s = (sem_s0, sem_s1)
    sem_i = (sem_i0, sem_i1, sem_i2, sem_i3)

    NG = E // CH

    def _off(i):
        g = jnp.minimum(wid + NW * i, NG - 1)
        return pl.multiple_of(g * CH, 8)

    def _fill_rows(ref, nrows, vec):
        def _f(i, _):
            for k in range(D // 16):
                ref[i, pl.ds(16 * k, 16)] = vec
            return 0
        lax.fori_loop(0, nrows, _f, 0)

    def _iload_sync(i, q, also_src):
        if also_src:
            pltpu.sync_copy(src_hbm.at[pl.ds(_off(i), CH)], idx_s[q])
        pltpu.sync_copy(dst_hbm.at[pl.ds(_off(i), CH)], idx_d[q])

    def _iload_async(i, q, also_src):
        if also_src:
            pltpu.async_copy(src_hbm.at[pl.ds(_off(i), CH)], idx_s[q], sem_i[q])
        pltpu.async_copy(dst_hbm.at[pl.ds(_off(i), CH)], idx_d[q], sem_i[q])

    def _iwait(q, also_src):
        pltpu.make_async_copy(dst_hbm.at[pl.ds(0, CH)], idx_d[q], sem_i[q]).wait()
        if also_src:
            pltpu.make_async_copy(src_hbm.at[pl.ds(0, CH)], idx_s[q], sem_i[q]).wait()

    def _gs(b, q):
        pltpu.async_copy(x_hbm.at[idx_s[q]], rows[b], sem_g[b])

    def _gd(b):
        pltpu.make_async_copy(x_hbm.at[idx_s[0]], rows[b], sem_g[b]).wait()

    def _ss(b, q, src_ref):
        pltpu.async_copy(src_ref, acc.at[idx_d[q]], sem_s[b], add=True)

    def _sd(b, src_ref):
        pltpu.make_async_copy(src_ref, acc.at[idx_d[0]], sem_s[b]).wait()

    def _zero_acc():
        for m in range(RPT // ZR):
            pltpu.async_copy(rows0.at[pl.ds(0, ZR)],
                             acc.at[pl.ds(rbase + m * ZR, ZR)], sem_w)
        for m in range(RPT // ZR):
            pltpu.make_async_copy(rows0.at[pl.ds(0, ZR)],
                                  acc.at[pl.ds(rbase, ZR)], sem_w).wait()

        @pl.when(s == NS - 1)
        def _zero_rem():
            pltpu.sync_copy(rows0.at[pl.ds(0, REM)], acc.at[pl.ds(NS * RPT, REM)])

    def _writeback(out_hbm):
        pltpu.async_copy(acc.at[pl.ds(rbase, RPT)],
                         out_hbm.at[c, pl.ds(rbase, RPT)], sem_w)

        @pl.when(s == NS - 1)
        def _wb_rem():
            pltpu.sync_copy(acc.at[pl.ds(NS * RPT, REM)],
                            out_hbm.at[c, pl.ds(NS * RPT, REM)])
        pltpu.make_async_copy(acc.at[pl.ds(rbase, RPT)],
                              out_hbm.at[c, pl.ds(rbase, RPT)], sem_w).wait()

    zero16 = jnp.zeros((16,), jnp.float32)
    one16 = jnp.ones((16,), jnp.float32)

    _fill_rows(rows0, ZR, zero16)
    _zero_acc()
    plsc.subcore_barrier()

    _iload_sync(0, 0, True)
    _iload_sync(1, 1, True)
    _iload_async(2, 2, True)
    _iload_async(3, 3, True)
    _gs(0, 0)
    _gs(1, 1)

    def _pair1(j, par):
        q2 = (2 * par + 2) % 4
        q3 = (2 * par + 3) % 4
        f0 = (2 * par) % 4
        f1 = (2 * par + 1) % 4
        _gd(0)
        _ss(0, f0, rows0)
        _gd(1)
        _ss(1, f1, rows1)
        _sd(0, rows0)
        _iload_async(j + 4, f0, True)
        _iwait(q2, True)
        _gs(0, q2)
        _sd(1, rows1)
        _iload_async(j + 5, f1, True)
        _iwait(q3, True)
        _gs(1, q3)

    def _quad1(q, _):
        _pair1(4 * q, 0)
        _pair1(4 * q + 2, 1)
        return 0
    lax.fori_loop(0, (NCHF - 2) // 4, _quad1, 0)

    _gd(0)
    _ss(0, 0, rows0)
    _gd(1)
    _ss(1, 1, rows1)
    _sd(0, rows0)
    _sd(1, rows1)

    @pl.when(wid < EXTRA)
    def _extra1():
        _iwait(2, True)
        _gs(0, 2)
        _gd(0)
        _ss(0, 2, rows0)
        _sd(0, rows0)

    @pl.when(wid >= EXTRA)
    def _drain2():
        _iwait(2, True)
    _iwait(3, True)
    plsc.subcore_barrier()

    _writeback(sum_out)
    plsc.subcore_barrier()

    _fill_rows(rows0, ZR, zero16)
    _zero_acc()
    plsc.subcore_barrier()
    _fill_rows(rows0, CH, one16)

    sems4 = (sem_s0, sem_s1, sem_g0, sem_g1)

    def _ss4(q, m):
        pltpu.async_copy(rows0, acc.at[idx_d[q]], sems4[m], add=True)

    def _sd4(m):
        pltpu.make_async_copy(rows0, acc.at[idx_d[0]], sems4[m]).wait()

    for b in range(4):
        _iload_sync(b, b, False)
        _ss4(b, b)

    def _quadB(qq, _):
        t0 = 4 * qq + 4
        for b in range(4):
            _sd4(b)
            _iload_sync(t0 + b, b, False)
            _ss4(b, b)
        return 0
    lax.fori_loop(0, (NCHF - 6) // 4, _quadB, 0)

    _sd4(0)
    _iload_sync(NCHF - 2, 0, False)
    _ss4(0, 0)
    _sd4(1)
    _iload_sync(NCHF - 1, 1, False)
    _ss4(1, 1)

    @pl.when(wid < EXTRA)
    def _extra2():
        _sd4(2)
        _iload_sync(NCHF, 2, False)
        _ss4(2, 2)
        _sd4(2)

    @pl.when(wid >= EXTRA)
    def _drain2b():
        _sd4(2)
    _sd4(3)
    _sd4(0)
    _sd4(1)
    plsc.subcore_barrier()

    _writeback(cnt_out)


@jax.jit
def _sc_aggregate(x, src, dst):
    mesh = plsc.VectorSubcoreMesh(core_axis_name="c", subcore_axis_name="s")
    idx_t = pltpu.VMEM((CH,), jnp.int32)
    dma = pltpu.SemaphoreType.DMA
    return pl.kernel(
        _sc_body,
        out_type=[
            jax.ShapeDtypeStruct((NC, N, D), jnp.float32),
            jax.ShapeDtypeStruct((NC, N, D), jnp.float32),
        ],
        mesh=mesh,
        scratch_types=[
            pltpu.VMEM_SHARED((N, D), jnp.float32),
            idx_t, idx_t, idx_t, idx_t, idx_t, idx_t, idx_t, idx_t,
            pltpu.VMEM((CH, D), jnp.float32),
            pltpu.VMEM((CH, D), jnp.float32),
            dma, dma, dma, dma, dma, dma, dma, dma, dma,
        ],
    )(x, src, dst)


def _tc_body(x_ref, sp_ref, cp_ref, ws_ref, wn_ref, bn_ref,
             w1_ref, b1_ref, w2_ref, b2_ref, o_ref):
    summed = sp_ref[0] + sp_ref[1]
    deg = cp_ref[0, :, 0:1] + cp_ref[1, :, 0:1]
    agg = summed / jnp.maximum(deg, 1.0)
    h = jnp.dot(x_ref[...], ws_ref[...], preferred_element_type=jnp.float32)
    h = h + jnp.dot(agg, wn_ref[...], preferred_element_type=jnp.float32)
    h = jnp.maximum(h + bn_ref[...], 0.0)
    h = jnp.maximum(
        jnp.dot(h, w1_ref[...], preferred_element_type=jnp.float32) + b1_ref[...], 0.0)
    o_ref[...] = (
        jnp.dot(h, w2_ref[...], preferred_element_type=jnp.float32) + b2_ref[...])


@jax.jit
def _tc_mlp(x, sum_p, cnt_p, W_self, W_neigh, b_neigh, W1, b1, W2, b2):
    B = 2000
    grid = (N // B,)
    wspec = pl.BlockSpec((128, 128), lambda i: (0, 0))
    bspec = pl.BlockSpec((1, 128), lambda i: (0, 0))
    return pl.pallas_call(
        _tc_body,
        grid=grid,
        in_specs=[
            pl.BlockSpec((B, D), lambda i: (i, 0)),
            pl.BlockSpec((NC, B, D), lambda i: (0, i, 0)),
            pl.BlockSpec((NC, B, D), lambda i: (0, i, 0)),
            wspec, wspec, bspec, wspec, bspec, wspec, bspec,
        ],
        out_specs=pl.BlockSpec((B, D), lambda i: (i, 0)),
        out_shape=jax.ShapeDtypeStruct((N, D), jnp.float32),
    )(x, sum_p, cnt_p, W_self, W_neigh, b_neigh, W1, b1, W2, b2)


def kernel(x, edge_index, W_self, W_neigh, b_neigh, W1, b1, W2, b2):
    src = edge_index[0].astype(jnp.int32)
    dst = edge_index[1].astype(jnp.int32)
    sum_p, cnt_p = _sc_aggregate(x, src, dst)
    return _tc_mlp(x, sum_p, cnt_p, W_self, W_neigh,
                   b_neigh.reshape(1, D), W1, b1.reshape(1, D),
                   W2, b2.reshape(1, D))


# speedup vs baseline: 1.0554x; 1.0554x over previous
import jax
import jax.numpy as jnp
from jax import lax
from jax.experimental import pallas as pl
from jax.experimental.pallas import tpu as pltpu
from jax.experimental.pallas import tpu_sc as plsc

N = 10000
E = 320000
D = 128

NC = 2
NS = 16
NW = NC * NS
EPW = E // NW
CH = 128
NCHF = E // (CH * NW)
EXTRA = (E // CH) - NCHF * NW
RPT = 624
REM = N - NS * RPT
ZR = 48


def _sc_body(x_hbm, edge_hbm, sum_out, cnt_out,
             acc, is0, is1, is2, is3, id0, id1, id2, id3, rows0, rows1,
             sem_g0, sem_g1, sem_s0, sem_s1, sem_i0, sem_i1, sem_i2, sem_i3,
             sem_w):
    c = lax.axis_index("c")
    s = lax.axis_index("s")
    wid = s * NC + c
    rbase = s * RPT

    idx_s = (is0, is1, is2, is3)
    idx_d = (id0, id1, id2, id3)
    rows = (rows0, rows1)
    sem_g = (sem_g0, sem_g1)
    sem_s = (sem_s0, sem_s1)
    sem_i = (sem_i0, sem_i1, sem_i2, sem_i3)

    NG = E // CH

    def _off(i):
        g = jnp.minimum(wid + NW * i, NG - 1)
        return pl.multiple_of(g * CH, 8)

    def _fill_rows(ref, nrows, vec):
        def _f(i, _):
            for k in range(D // 16):
                ref[i, pl.ds(16 * k, 16)] = vec
            return 0
        lax.fori_loop(0, nrows, _f, 0)

    def _iload_sync(i, q, also_src):
        if also_src:
            pltpu.sync_copy(edge_hbm.at[0, pl.ds(_off(i), CH)], idx_s[q])
        pltpu.sync_copy(edge_hbm.at[1, pl.ds(_off(i), CH)], idx_d[q])

    def _iload_async(i, q, also_src):
        if also_src:
            pltpu.async_copy(edge_hbm.at[0, pl.ds(_off(i), CH)], idx_s[q], sem_i[q])
        pltpu.async_copy(edge_hbm.at[1, pl.ds(_off(i), CH)], idx_d[q], sem_i[q])

    def _iwait(q, also_src):
        pltpu.make_async_copy(edge_hbm.at[1, pl.ds(0, CH)], idx_d[q], sem_i[q]).wait()
        if also_src:
            pltpu.make_async_copy(edge_hbm.at[0, pl.ds(0, CH)], idx_s[q], sem_i[q]).wait()

    def _gs(b, q):
        pltpu.async_copy(x_hbm.at[idx_s[q]], rows[b], sem_g[b])

    def _gd(b):
        pltpu.make_async_copy(x_hbm.at[idx_s[0]], rows[b], sem_g[b]).wait()

    def _ss(b, q, src_ref):
        pltpu.async_copy(src_ref, acc.at[idx_d[q]], sem_s[b], add=True)

    def _sd(b, src_ref):
        pltpu.make_async_copy(src_ref, acc.at[idx_d[0]], sem_s[b]).wait()

    def _zero_acc():
        for m in range(RPT // ZR):
            pltpu.async_copy(rows0.at[pl.ds(0, ZR)],
                             acc.at[pl.ds(rbase + m * ZR, ZR)], sem_w)
        for m in range(RPT // ZR):
            pltpu.make_async_copy(rows0.at[pl.ds(0, ZR)],
                                  acc.at[pl.ds(rbase, ZR)], sem_w).wait()

        @pl.when(s == NS - 1)
        def _zero_rem():
            pltpu.sync_copy(rows0.at[pl.ds(0, REM)], acc.at[pl.ds(NS * RPT, REM)])

    def _writeback(out_hbm):
        pltpu.async_copy(acc.at[pl.ds(rbase, RPT)],
                         out_hbm.at[c, pl.ds(rbase, RPT)], sem_w)

        @pl.when(s == NS - 1)
        def _wb_rem():
            pltpu.sync_copy(acc.at[pl.ds(NS * RPT, REM)],
                            out_hbm.at[c, pl.ds(NS * RPT, REM)])
        pltpu.make_async_copy(acc.at[pl.ds(rbase, RPT)],
                              out_hbm.at[c, pl.ds(rbase, RPT)], sem_w).wait()

    zero16 = jnp.zeros((16,), jnp.float32)
    one16 = jnp.ones((16,), jnp.float32)

    _fill_rows(rows0, ZR, zero16)
    _zero_acc()
    plsc.subcore_barrier()

    _iload_sync(0, 0, True)
    _iload_sync(1, 1, True)
    _iload_async(2, 2, True)
    _iload_async(3, 3, True)
    _gs(0, 0)
    _gs(1, 1)

    def _pair1(j, par):
        q2 = (2 * par + 2) % 4
        q3 = (2 * par + 3) % 4
        f0 = (2 * par) % 4
        f1 = (2 * par + 1) % 4
        _gd(0)
        _ss(0, f0, rows0)
        _gd(1)
        _ss(1, f1, rows1)
        _sd(0, rows0)
        _iload_async(j + 4, f0, True)
        _iwait(q2, True)
        _gs(0, q2)
        _sd(1, rows1)
        _iload_async(j + 5, f1, True)
        _iwait(q3, True)
        _gs(1, q3)

    def _quad1(q, _):
        _pair1(4 * q, 0)
        _pair1(4 * q + 2, 1)
        return 0
    lax.fori_loop(0, (NCHF - 2) // 4, _quad1, 0)

    _gd(0)
    _ss(0, 0, rows0)
    _gd(1)
    _ss(1, 1, rows1)
    _sd(0, rows0)
    _sd(1, rows1)

    @pl.when(wid < EXTRA)
    def _extra1():
        _iwait(2, True)
        _gs(0, 2)
        _gd(0)
        _ss(0, 2, rows0)
        _sd(0, rows0)

    @pl.when(wid >= EXTRA)
    def _drain2():
        _iwait(2, True)
    _iwait(3, True)
    plsc.subcore_barrier()

    _writeback(sum_out)
    plsc.subcore_barrier()

    _fill_rows(rows0, ZR, zero16)
    _zero_acc()
    plsc.subcore_barrier()
    _fill_rows(rows0, CH, one16)

    sems4 = (sem_s0, sem_s1, sem_g0, sem_g1)

    def _ss4(q, m):
        pltpu.async_copy(rows0, acc.at[idx_d[q]], sems4[m], add=True)

    def _sd4(m):
        pltpu.make_async_copy(rows0, acc.at[idx_d[0]], sems4[m]).wait()

    for b in range(4):
        _iload_sync(b, b, False)
        _ss4(b, b)

    def _quadB(qq, _):
        t0 = 4 * qq + 4
        for b in range(4):
            _sd4(b)
            _iload_sync(t0 + b, b, False)
            _ss4(b, b)
        return 0
    lax.fori_loop(0, (NCHF - 6) // 4, _quadB, 0)

    _sd4(0)
    _iload_sync(NCHF - 2, 0, False)
    _ss4(0, 0)
    _sd4(1)
    _iload_sync(NCHF - 1, 1, False)
    _ss4(1, 1)

    @pl.when(wid < EXTRA)
    def _extra2():
        _sd4(2)
        _iload_sync(NCHF, 2, False)
        _ss4(2, 2)
        _sd4(2)

    @pl.when(wid >= EXTRA)
    def _drain2b():
        _sd4(2)
    _sd4(3)
    _sd4(0)
    _sd4(1)
    plsc.subcore_barrier()

    _writeback(cnt_out)


@jax.jit
def _sc_aggregate(x, edge_index):
    mesh = plsc.VectorSubcoreMesh(core_axis_name="c", subcore_axis_name="s")
    idx_t = pltpu.VMEM((CH,), jnp.int32)
    dma = pltpu.SemaphoreType.DMA
    return pl.kernel(
        _sc_body,
        out_type=[
            jax.ShapeDtypeStruct((NC, N, D), jnp.float32),
            jax.ShapeDtypeStruct((NC, N, D), jnp.float32),
        ],
        mesh=mesh,
        scratch_types=[
            pltpu.VMEM_SHARED((N, D), jnp.float32),
            idx_t, idx_t, idx_t, idx_t, idx_t, idx_t, idx_t, idx_t,
            pltpu.VMEM((CH, D), jnp.float32),
            pltpu.VMEM((CH, D), jnp.float32),
            dma, dma, dma, dma, dma, dma, dma, dma, dma,
        ],
    )(x, edge_index)


def _tc_body(x_ref, sp_ref, cp_ref, ws_ref, wn_ref, bn_ref,
             w1_ref, b1_ref, w2_ref, b2_ref, o_ref):
    summed = sp_ref[0] + sp_ref[1]
    deg = cp_ref[0, :, 0:1] + cp_ref[1, :, 0:1]
    agg = summed / jnp.maximum(deg, 1.0)
    h = jnp.dot(x_ref[...], ws_ref[...], preferred_element_type=jnp.float32)
    h = h + jnp.dot(agg, wn_ref[...], preferred_element_type=jnp.float32)
    h = jnp.maximum(h + bn_ref[...], 0.0)
    h = jnp.maximum(
        jnp.dot(h, w1_ref[...], preferred_element_type=jnp.float32) + b1_ref[...], 0.0)
    o_ref[...] = (
        jnp.dot(h, w2_ref[...], preferred_element_type=jnp.float32) + b2_ref[...])


@jax.jit
def _tc_mlp(x, sum_p, cnt_p, W_self, W_neigh, b_neigh, W1, b1, W2, b2):
    B = 2000
    grid = (N // B,)
    wspec = pl.BlockSpec((128, 128), lambda i: (0, 0))
    bspec = pl.BlockSpec((1, 128), lambda i: (0, 0))
    return pl.pallas_call(
        _tc_body,
        grid=grid,
        in_specs=[
            pl.BlockSpec((B, D), lambda i: (i, 0)),
            pl.BlockSpec((NC, B, D), lambda i: (0, i, 0)),
            pl.BlockSpec((NC, B, D), lambda i: (0, i, 0)),
            wspec, wspec, bspec, wspec, bspec, wspec, bspec,
        ],
        out_specs=pl.BlockSpec((B, D), lambda i: (i, 0)),
        out_shape=jax.ShapeDtypeStruct((N, D), jnp.float32),
    )(x, sum_p, cnt_p, W_self, W_neigh, b_neigh, W1, b1, W2, b2)


def kernel(x, edge_index, W_self, W_neigh, b_neigh, W1, b1, W2, b2):
    sum_p, cnt_p = _sc_aggregate(x, edge_index.astype(jnp.int32))
    return _tc_mlp(x, sum_p, cnt_p, W_self, W_neigh,
                   b_neigh.reshape(1, D), W1, b1.reshape(1, D),
                   W2, b2.reshape(1, D))
